# Initial kernel scaffold; baseline (speedup 1.0000x reference)
import jax
import jax.numpy as jnp
from jax.experimental import pallas as pl


def kernel(z, A, z0):
    raise NotImplementedError("write your pallas kernel here")


# speedup vs baseline: 2.2650x; 2.2650x over previous
import functools

import jax
import jax.numpy as jnp
from jax import lax
from jax.experimental import pallas as pl
from jax.experimental.pallas import tpu as pltpu
from jax.experimental.pallas import tpu_sc as plsc


def _argmax_body(z_ref, sel_ref):
    z = z_ref[...]
    b, k = z.shape
    m = jnp.max(z, axis=1, keepdims=True)
    iota = lax.broadcasted_iota(jnp.int32, (b, k), 1)
    cand = jnp.where(z == m, iota, k)
    sel_ref[...] = jnp.min(cand, axis=1, keepdims=True)


def _argmax_tc(z, block_rows=2048):
    n, k = z.shape
    return pl.pallas_call(
        _argmax_body,
        grid=(n // block_rows,),
        in_specs=[pl.BlockSpec((block_rows, k), lambda j: (j, 0))],
        out_specs=pl.BlockSpec((block_rows, 1), lambda j: (j, 0)),
        out_shape=jax.ShapeDtypeStruct((n, 1), jnp.int32),
    )(z)


def _gather_sc(table, idx1d, n, k):
    info = plsc.get_sparse_core_info()
    nc, ns = info.num_cores, info.num_subcores
    nw = nc * ns
    rows_per_w = n // nw
    chunk = 512
    n_chunks = rows_per_w // chunk
    g_per_chunk = chunk // 128
    mesh = plsc.VectorSubcoreMesh(core_axis_name="c", subcore_axis_name="s")

    @functools.partial(
        pl.kernel,
        mesh=mesh,
        out_type=jax.ShapeDtypeStruct((n, k), jnp.float32),
        scratch_types=[
            pltpu.VMEM((chunk,), jnp.int32),
            pltpu.VMEM((chunk, k), jnp.float32),
            pltpu.SemaphoreType.DMA,
        ],
    )
    def gather_kernel(table_hbm, idx_hbm, out_hbm, idx_v, rows_v, sem):
        wid = lax.axis_index("s") * nc + lax.axis_index("c")
        row0 = wid * rows_per_w

        def body(i, carry):
            off = row0 + i * chunk
            pltpu.sync_copy(idx_hbm.at[pl.ds(off, chunk)], idx_v)
            for j in range(g_per_chunk):
                pltpu.async_copy(
                    table_hbm.at[idx_v.at[pl.ds(j * 128, 128)]],
                    rows_v.at[pl.ds(j * 128, 128)],
                    sem,
                ).wait()
            pltpu.sync_copy(rows_v, out_hbm.at[pl.ds(off, chunk)])
            return carry

        lax.fori_loop(0, n_chunks, body, 0)

    return gather_kernel(table, idx1d)


def kernel(z, A, z0):
    n, k = z.shape
    sel = _argmax_tc(z).reshape(n)
    idx = jnp.concatenate([jnp.full((1,), k, jnp.int32), sel[:-1]])
    table = jnp.concatenate([A, z0.astype(A.dtype)], axis=0)
    return _gather_sc(table, idx, n, k)
